# baseline (speedup 1.0000x reference)
import functools

import jax
import jax.numpy as jnp
import numpy as np
from jax import lax
from jax.experimental import pallas as pl
from jax.experimental.pallas import tpu as pltpu
from jax.experimental.pallas import tpu_sc as plsc

jax.config.update("jax_enable_x64", True)


def _i0():
    return jnp.int32(0)

H = 8
TP = 2
N = 2 ** H
K = (2 * H) // 2
VOCAB = 512
BATCH = 128
D = 256
NCNC = 2 * H * K + 1
CPAD = 136
NB = VOCAB // BATCH


def _rng_constants():
    key = jax.random.key(42)
    perm = jax.random.permutation(jax.random.fold_in(key, 0), VOCAB)
    upper = (2 ** jnp.arange(H, dtype=jnp.int64)).reshape(-1, 1, 1, 1)
    flips = (2 ** jnp.arange(H, dtype=jnp.int64))
    perm_out = perm.reshape(NB, BATCH).astype(jnp.int32)
    g0s, g1s, ss = [], [], []
    for b in range(NB):
        kb = jax.random.fold_in(key, b + 1)
        km, kp = jax.random.split(kb)
        rnd = jax.random.randint(km, (H, BATCH, K, TP), 0, N, dtype=jnp.int64)
        masks = jnp.transpose(rnd % upper, (1, 0, 2, 3))
        g_res = (flips[None, :, None, None] ^ masks).reshape(BATCH, H * K, TP)
        g_full = jnp.concatenate(
            [g_res, jnp.zeros((BATCH, 1, TP), jnp.int64), g_res], axis=1)
        s_full = jnp.concatenate(
            [jnp.ones(H * K + 1, jnp.float32), -jnp.ones(H * K, jnp.float32)])
        pidx = jax.random.permutation(kp, NCNC)
        g_p = g_full[:, pidx, :].astype(jnp.int32)
        s_p = s_full[pidx]
        g_pad = jnp.concatenate(
            [g_p, jnp.zeros((BATCH, CPAD - NCNC, TP), jnp.int32)], axis=1)
        s_pad = jnp.concatenate(
            [s_p, jnp.ones((CPAD - NCNC,), jnp.float32)])
        g0s.append(g_pad[:, :, 0:1])
        g1s.append(g_pad[:, :, 1:2])
        ss.append(s_pad.reshape(CPAD, 1))
    return (perm_out, jnp.stack(g0s), jnp.stack(g1s), jnp.stack(ss))


NWORK = 32
RPW = VOCAB // NWORK
GRAN = 128


def _sc_gather_kernel(table_hbm, idx_hbm, out_hbm, idx_v, rows_v, sem):
    wid = lax.axis_index("s") * 2 + lax.axis_index("c")
    base = wid * RPW
    pltpu.sync_copy(idx_hbm.at[pl.ds(base, RPW)], idx_v)
    pltpu.async_copy(table_hbm.at[idx_v], rows_v, sem).wait()
    pltpu.sync_copy(rows_v, out_hbm.at[pl.ds(base, RPW)])


def _sc_scatter_kernel(rows_hbm, idx_hbm, out_hbm, idx_v, rows_v, sem):
    wid = lax.axis_index("s") * 2 + lax.axis_index("c")
    base = wid * RPW
    pltpu.sync_copy(idx_hbm.at[pl.ds(base, RPW)], idx_v)
    pltpu.sync_copy(rows_hbm.at[pl.ds(base, RPW)], rows_v)
    pltpu.async_copy(rows_v, out_hbm.at[idx_v], sem).wait()


def _sc_call(body, n_out_rows):
    return pl.kernel(
        body,
        mesh=plsc.VectorSubcoreMesh(core_axis_name="c", subcore_axis_name="s"),
        out_type=jax.ShapeDtypeStruct((n_out_rows, GRAN), jnp.int32),
        scratch_types=[
            pltpu.VMEM((RPW,), jnp.int32),
            pltpu.VMEM((RPW, GRAN), jnp.int32),
            pltpu.SemaphoreType.DMA,
        ],
    )


def _logits_kernel(a_ref, out_ref):
    out_ref[...] = jax.lax.dot_general(
        a_ref[...], a_ref[...], (((1,), (1,)), ((), ())),
        preferred_element_type=jnp.float32)


def _exp_field(x_i32):
    y = (x_i32 + 1).astype(jnp.float32)
    return jax.lax.shift_right_arithmetic(
        jax.lax.bitcast_convert_type(y, jnp.int32), jnp.int32(23))


def _main_kernel(perm_ref, sta_smem, g0_ref, g1_ref, s_ref, logits_ref,
                 loct_in_ref, loc_out_ref, tl_ref,
                 loct, ap, spv, pend_cov, pend_upd, acc_tl):
    g = pl.program_id(0)
    r = pl.program_id(1)

    @pl.when(jnp.logical_and(g == 0, r == 0))
    def _init():
        loct[...] = loct_in_ref[...].astype(jnp.float32)
        acc_tl[...] = jnp.zeros_like(acc_tl)

    @pl.when(r == 0)
    def _batch_start():
        ap[...] = jnp.abs(loct[...]).astype(jnp.int32)
        spv[...] = jnp.sign(loct[...]) * jnp.float32(1.0 / 16.0)
        pend_cov[...] = jnp.zeros_like(pend_cov)
        pend_upd[...] = jnp.zeros_like(pend_upd)

    sub_iota = jax.lax.broadcasted_iota(
        jnp.int32, (CPAD, 1), 0).astype(jnp.float32)
    lane_iota = jax.lax.broadcasted_iota(jnp.int32, (1, VOCAB), 1)
    s_col = s_ref[0]

    idx = perm_ref[g, r]
    l0 = sta_smem[g, r, 0]
    l1 = sta_smem[g, r, 1]
    logit_row = logits_ref[pl.ds(idx, 1), :]
    ori = (jnp.abs(l0), jnp.abs(l1))
    sgn = (jnp.sign(l0).astype(jnp.float32), jnp.sign(l1).astype(jnp.float32))
    g_rows = (jnp.reshape(g0_ref[0, 0], (CPAD, 1)),
              jnp.reshape(g1_ref[0, 0], (CPAD, 1)))
    lane_mask = (lane_iota == idx).astype(jnp.float32)
    pend_cov[...] += lane_mask
    for t in (0, 1):
        o = 1 - t
        ap_t = ap[pl.ds(t, 1), :]
        ap_o = ap[pl.ds(o, 1), :]
        spv_t = spv[pl.ds(t, 1), :]
        spv_o = spv[pl.ds(o, 1), :]
        e2 = _exp_field(ori[o] ^ ap_o)
        dsph = (134 - e2).astype(jnp.float32) * (spv_o * sgn[o])
        u = ori[t] ^ g_rows[t]
        gj = jnp.where(u == 0, jnp.float32(0.0), s_col)
        e = _exp_field(u ^ ap_t)
        dh = (134 - e).astype(jnp.float32) * (gj * spv_t)
        delt = (dh + dsph) - logit_row
        loss = jnp.sum(jnp.abs(delt), axis=1, keepdims=True)
        loss = jnp.where(sub_iota < NCNC, loss, jnp.float32(3.0e38))
        m = jnp.min(loss, axis=0, keepdims=True)
        jstar = jnp.min(jnp.where(loss == m, sub_iota, jnp.float32(1.0e9)),
                        axis=0, keepdims=True)
        candv = s_col * u.astype(jnp.float32)
        sel = jnp.sum(jnp.where(sub_iota == jstar, candv, jnp.float32(0.0)),
                      axis=0, keepdims=True)
        acc_tl[...] += m
        loc_out_ref[0, pl.ds(0, 1), pl.ds(t, 1)] = sel.astype(jnp.int32)
        pend_upd[pl.ds(t, 1), :] += lane_mask * sel

    @pl.when(r == BATCH - 1)
    def _batch_end():
        loct[...] = (loct[...] * (jnp.float32(1.0) - pend_cov[...])
                     + pend_upd[...])

    @pl.when(jnp.logical_and(g == NB - 1, r == BATCH - 1))
    def _fin():
        tl_ref[...] = acc_tl[...] * jnp.float32(1.0 / (VOCAB * BATCH * TP * NB))


def kernel(eu_emb, locations):
    eu32 = eu_emb.astype(jnp.float32)
    loc32 = locations.astype(jnp.int32)
    loct_in = loc32.T
    perm_c, g0_c, g1_c, s_c = _rng_constants()
    perm_flat = perm_c.reshape(VOCAB)

    loc_pad = jnp.pad(loc32, ((0, 0), (0, GRAN - TP)))
    sta_pad = _sc_call(_sc_gather_kernel, VOCAB)(loc_pad, perm_flat)
    sta_c = sta_pad[:, :TP].reshape(NB, BATCH, TP)

    logits = pl.pallas_call(
        _logits_kernel,
        out_shape=jax.ShapeDtypeStruct((VOCAB, VOCAB), jnp.float32),
    )(eu32)

    grid_spec = pltpu.PrefetchScalarGridSpec(
        num_scalar_prefetch=0,
        grid=(NB, BATCH),
        in_specs=[
            pl.BlockSpec((NB, BATCH), lambda g, r: (_i0(), _i0()),
                         memory_space=pltpu.SMEM),
            pl.BlockSpec((NB, BATCH, TP), lambda g, r: (_i0(), _i0(), _i0()),
                         memory_space=pltpu.SMEM),
            pl.BlockSpec((1, 1, CPAD, 1), lambda g, r: (g, r, _i0(), _i0())),
            pl.BlockSpec((1, 1, CPAD, 1), lambda g, r: (g, r, _i0(), _i0())),
            pl.BlockSpec((1, CPAD, 1), lambda g, r: (g, _i0(), _i0())),
            pl.BlockSpec((VOCAB, VOCAB), lambda g, r: (_i0(), _i0())),
            pl.BlockSpec((TP, VOCAB), lambda g, r: (_i0(), _i0())),
        ],
        out_specs=[
            pl.BlockSpec((1, 1, TP), lambda g, r: (g * BATCH + r, _i0(), _i0())),
            pl.BlockSpec((1, 1), lambda g, r: (_i0(), _i0())),
        ],
        scratch_shapes=[
            pltpu.VMEM((TP, VOCAB), jnp.float32),
            pltpu.VMEM((TP, VOCAB), jnp.int32),
            pltpu.VMEM((TP, VOCAB), jnp.float32),
            pltpu.VMEM((1, VOCAB), jnp.float32),
            pltpu.VMEM((TP, VOCAB), jnp.float32),
            pltpu.VMEM((1, 1), jnp.float32),
        ],
    )

    sel_seq, tl = pl.pallas_call(
        _main_kernel,
        grid_spec=grid_spec,
        out_shape=[
            jax.ShapeDtypeStruct((NB * BATCH, 1, TP), jnp.int32),
            jax.ShapeDtypeStruct((1, 1), jnp.float32),
        ],
    )(perm_c, sta_c, g0_c, g1_c, s_c, logits, loct_in)

    sel_pad = jnp.pad(sel_seq.reshape(VOCAB, TP), ((0, 0), (0, GRAN - TP)))
    out_pad = _sc_call(_sc_scatter_kernel, VOCAB)(sel_pad, perm_flat)
    loc_out = out_pad[:, :TP]

    return loc_out.astype(locations.dtype), tl.reshape(()).astype(jnp.float32)


# speedup vs baseline: 1.6079x; 1.6079x over previous
import functools

import jax
import jax.numpy as jnp
import numpy as np
from jax import lax
from jax.experimental import pallas as pl
from jax.experimental.pallas import tpu as pltpu
from jax.experimental.pallas import tpu_sc as plsc

jax.config.update("jax_enable_x64", True)


def _i0():
    return jnp.int32(0)

H = 8
TP = 2
N = 2 ** H
K = (2 * H) // 2
VOCAB = 512
BATCH = 128
D = 256
NCNC = 2 * H * K + 1
CPAD = 136
NB = VOCAB // BATCH


def _rng_constants():
    key = jax.random.key(42)
    perm = jax.random.permutation(jax.random.fold_in(key, 0), VOCAB)
    upper = (2 ** jnp.arange(H, dtype=jnp.int64)).reshape(-1, 1, 1, 1)
    flips = (2 ** jnp.arange(H, dtype=jnp.int64))
    perm_out = perm.reshape(NB, BATCH).astype(jnp.int32)
    g0s, g1s, ss = [], [], []
    for b in range(NB):
        kb = jax.random.fold_in(key, b + 1)
        km, kp = jax.random.split(kb)
        rnd = jax.random.randint(km, (H, BATCH, K, TP), 0, N, dtype=jnp.int64)
        masks = jnp.transpose(rnd % upper, (1, 0, 2, 3))
        g_res = (flips[None, :, None, None] ^ masks).reshape(BATCH, H * K, TP)
        g_full = jnp.concatenate(
            [g_res, jnp.zeros((BATCH, 1, TP), jnp.int64), g_res], axis=1)
        s_full = jnp.concatenate(
            [jnp.ones(H * K + 1, jnp.float32), -jnp.ones(H * K, jnp.float32)])
        pidx = jax.random.permutation(kp, NCNC)
        g_p = g_full[:, pidx, :].astype(jnp.int32)
        s_p = s_full[pidx]
        g_pad = jnp.concatenate(
            [g_p, jnp.zeros((BATCH, CPAD - NCNC, TP), jnp.int32)], axis=1)
        s_pad = jnp.concatenate(
            [s_p, jnp.ones((CPAD - NCNC,), jnp.float32)])
        g0s.append(g_pad[:, :, 0:1])
        g1s.append(g_pad[:, :, 1:2])
        ss.append(s_pad.reshape(CPAD, 1))
    return (perm_out, jnp.stack(g0s), jnp.stack(g1s), jnp.stack(ss))


NWORK = 32
RPW = VOCAB // NWORK
GRAN = 128


def _sc_gather_kernel(table_hbm, idx_hbm, out_hbm, idx_v, rows_v, sem):
    wid = lax.axis_index("s") * 2 + lax.axis_index("c")
    base = wid * RPW
    pltpu.sync_copy(idx_hbm.at[pl.ds(base, RPW)], idx_v)
    pltpu.async_copy(table_hbm.at[idx_v], rows_v, sem).wait()
    pltpu.sync_copy(rows_v, out_hbm.at[pl.ds(base, RPW)])


def _sc_scatter_kernel(rows_hbm, idx_hbm, out_hbm, idx_v, rows_v, sem):
    wid = lax.axis_index("s") * 2 + lax.axis_index("c")
    base = wid * RPW
    pltpu.sync_copy(idx_hbm.at[pl.ds(base, RPW)], idx_v)
    pltpu.sync_copy(rows_hbm.at[pl.ds(base, RPW)], rows_v)
    pltpu.async_copy(rows_v, out_hbm.at[idx_v], sem).wait()


def _sc_call(body, n_out_rows):
    return pl.kernel(
        body,
        mesh=plsc.VectorSubcoreMesh(core_axis_name="c", subcore_axis_name="s"),
        out_type=jax.ShapeDtypeStruct((n_out_rows, GRAN), jnp.int32),
        scratch_types=[
            pltpu.VMEM((RPW,), jnp.int32),
            pltpu.VMEM((RPW, GRAN), jnp.int32),
            pltpu.SemaphoreType.DMA,
        ],
    )


def _logits_kernel(a_ref, out_ref):
    out_ref[...] = jax.lax.dot_general(
        a_ref[...], a_ref[...], (((1,), (1,)), ((), ())),
        preferred_element_type=jnp.float32)


def _exp_field(x_i32):
    y = (x_i32 + 1).astype(jnp.float32)
    return jax.lax.shift_right_arithmetic(
        jax.lax.bitcast_convert_type(y, jnp.int32), jnp.int32(23))


def _main_kernel(perm_ref, sta_smem, g0_ref, g1_ref, s_ref, logits_ref,
                 loct_in_ref, loc_out_ref, tl_ref,
                 loct, ap, spv, pend_cov, pend_upd, acc_tl):
    g = pl.program_id(0)
    r = pl.program_id(1)

    @pl.when(jnp.logical_and(g == 0, r == 0))
    def _init():
        loct[...] = loct_in_ref[...].astype(jnp.float32)
        acc_tl[...] = jnp.zeros_like(acc_tl)

    @pl.when(r == 0)
    def _batch_start():
        ap[...] = jnp.abs(loct[...]).astype(jnp.int32)
        spv[...] = jnp.sign(loct[...]) * jnp.float32(1.0 / 16.0)
        pend_cov[...] = jnp.zeros_like(pend_cov)
        pend_upd[...] = jnp.zeros_like(pend_upd)

    sub_iota = jax.lax.broadcasted_iota(
        jnp.int32, (CPAD, 1), 0).astype(jnp.float32)
    lane_iota = jax.lax.broadcasted_iota(jnp.int32, (1, VOCAB), 1)
    s_col = s_ref[0]

    idx = perm_ref[g, r]
    l0 = sta_smem[g, r, 0]
    l1 = sta_smem[g, r, 1]
    logit_row = logits_ref[pl.ds(idx, 1), :]
    ori = (jnp.abs(l0), jnp.abs(l1))
    sgn = (jnp.sign(l0).astype(jnp.float32), jnp.sign(l1).astype(jnp.float32))
    g_rows = (jnp.reshape(g0_ref[0, 0], (CPAD, 1)),
              jnp.reshape(g1_ref[0, 0], (CPAD, 1)))
    lane_mask = (lane_iota == idx).astype(jnp.float32)
    pend_cov[...] += lane_mask
    for t in (0, 1):
        o = 1 - t
        ap_t = ap[pl.ds(t, 1), :]
        ap_o = ap[pl.ds(o, 1), :]
        spv_t = spv[pl.ds(t, 1), :]
        spv_o = spv[pl.ds(o, 1), :]
        e2 = _exp_field(ori[o] ^ ap_o)
        dsph = (134 - e2).astype(jnp.float32) * (spv_o * sgn[o])
        u = ori[t] ^ g_rows[t]
        gj = jnp.where(u == 0, jnp.float32(0.0), s_col)
        e = _exp_field(u ^ ap_t)
        dh = (134 - e).astype(jnp.float32) * (gj * spv_t)
        delt = (dh + dsph) - logit_row
        loss = jnp.sum(jnp.abs(delt), axis=1, keepdims=True)
        loss = jnp.where(sub_iota < NCNC, loss, jnp.float32(3.0e38))
        m = jnp.min(loss, axis=0, keepdims=True)
        jstar = jnp.min(jnp.where(loss == m, sub_iota, jnp.float32(1.0e9)),
                        axis=0, keepdims=True)
        candv = s_col * u.astype(jnp.float32)
        sel = jnp.sum(jnp.where(sub_iota == jstar, candv, jnp.float32(0.0)),
                      axis=0, keepdims=True)
        acc_tl[...] += m
        loc_out_ref[0, pl.ds(0, 1), pl.ds(t, 1)] = sel.astype(jnp.int32)
        pend_upd[pl.ds(t, 1), :] += lane_mask * sel

    @pl.when(r == BATCH - 1)
    def _batch_end():
        loct[...] = (loct[...] * (jnp.float32(1.0) - pend_cov[...])
                     + pend_upd[...])

    @pl.when(jnp.logical_and(g == NB - 1, r == BATCH - 1))
    def _fin():
        tl_ref[...] = acc_tl[...] * jnp.float32(1.0 / (VOCAB * BATCH * TP * NB))


def kernel(eu_emb, locations):
    eu32 = eu_emb.astype(jnp.float32)
    loc32 = locations.astype(jnp.int32)
    loct_in = loc32.T
    with jax.ensure_compile_time_eval():
        perm_c, g0_c, g1_c, s_c = _rng_constants()
    perm_flat = perm_c.reshape(VOCAB)

    loc_pad = jnp.pad(loc32, ((0, 0), (0, GRAN - TP)))
    sta_pad = _sc_call(_sc_gather_kernel, VOCAB)(loc_pad, perm_flat)
    sta_c = sta_pad[:, :TP].reshape(NB, BATCH, TP)

    logits = pl.pallas_call(
        _logits_kernel,
        out_shape=jax.ShapeDtypeStruct((VOCAB, VOCAB), jnp.float32),
    )(eu32)

    grid_spec = pltpu.PrefetchScalarGridSpec(
        num_scalar_prefetch=0,
        grid=(NB, BATCH),
        in_specs=[
            pl.BlockSpec((NB, BATCH), lambda g, r: (_i0(), _i0()),
                         memory_space=pltpu.SMEM),
            pl.BlockSpec((NB, BATCH, TP), lambda g, r: (_i0(), _i0(), _i0()),
                         memory_space=pltpu.SMEM),
            pl.BlockSpec((1, 1, CPAD, 1), lambda g, r: (g, r, _i0(), _i0())),
            pl.BlockSpec((1, 1, CPAD, 1), lambda g, r: (g, r, _i0(), _i0())),
            pl.BlockSpec((1, CPAD, 1), lambda g, r: (g, _i0(), _i0())),
            pl.BlockSpec((VOCAB, VOCAB), lambda g, r: (_i0(), _i0())),
            pl.BlockSpec((TP, VOCAB), lambda g, r: (_i0(), _i0())),
        ],
        out_specs=[
            pl.BlockSpec((1, 1, TP), lambda g, r: (g * BATCH + r, _i0(), _i0())),
            pl.BlockSpec((1, 1), lambda g, r: (_i0(), _i0())),
        ],
        scratch_shapes=[
            pltpu.VMEM((TP, VOCAB), jnp.float32),
            pltpu.VMEM((TP, VOCAB), jnp.int32),
            pltpu.VMEM((TP, VOCAB), jnp.float32),
            pltpu.VMEM((1, VOCAB), jnp.float32),
            pltpu.VMEM((TP, VOCAB), jnp.float32),
            pltpu.VMEM((1, 1), jnp.float32),
        ],
    )

    sel_seq, tl = pl.pallas_call(
        _main_kernel,
        grid_spec=grid_spec,
        out_shape=[
            jax.ShapeDtypeStruct((NB * BATCH, 1, TP), jnp.int32),
            jax.ShapeDtypeStruct((1, 1), jnp.float32),
        ],
    )(perm_c, sta_c, g0_c, g1_c, s_c, logits, loct_in)

    sel_pad = jnp.pad(sel_seq.reshape(VOCAB, TP), ((0, 0), (0, GRAN - TP)))
    out_pad = _sc_call(_sc_scatter_kernel, VOCAB)(sel_pad, perm_flat)
    loc_out = out_pad[:, :TP]

    return loc_out.astype(locations.dtype), tl.reshape(()).astype(jnp.float32)


# speedup vs baseline: 2.1549x; 1.3402x over previous
import functools

import jax
import jax.numpy as jnp
import numpy as np
from jax import lax
from jax.experimental import pallas as pl
from jax.experimental.pallas import tpu as pltpu
from jax.experimental.pallas import tpu_sc as plsc

jax.config.update("jax_enable_x64", True)


def _i0():
    return jnp.int32(0)

H = 8
TP = 2
N = 2 ** H
K = (2 * H) // 2
VOCAB = 512
BATCH = 128
D = 256
NCNC = 2 * H * K + 1
CPAD = 136
NB = VOCAB // BATCH
RB = 4


def _rng_constants():
    key = jax.random.key(42)
    perm = jax.random.permutation(jax.random.fold_in(key, 0), VOCAB)
    upper = (2 ** jnp.arange(H, dtype=jnp.int64)).reshape(-1, 1, 1, 1)
    flips = (2 ** jnp.arange(H, dtype=jnp.int64))
    perm_out = perm.reshape(NB, BATCH).astype(jnp.int32)
    g0s, g1s, ss = [], [], []
    for b in range(NB):
        kb = jax.random.fold_in(key, b + 1)
        km, kp = jax.random.split(kb)
        rnd = jax.random.randint(km, (H, BATCH, K, TP), 0, N, dtype=jnp.int64)
        masks = jnp.transpose(rnd % upper, (1, 0, 2, 3))
        g_res = (flips[None, :, None, None] ^ masks).reshape(BATCH, H * K, TP)
        g_full = jnp.concatenate(
            [g_res, jnp.zeros((BATCH, 1, TP), jnp.int64), g_res], axis=1)
        s_full = jnp.concatenate(
            [jnp.ones(H * K + 1, jnp.float32), -jnp.ones(H * K, jnp.float32)])
        pidx = jax.random.permutation(kp, NCNC)
        g_p = g_full[:, pidx, :].astype(jnp.int32)
        s_p = s_full[pidx]
        g_pad = jnp.concatenate(
            [g_p, jnp.zeros((BATCH, CPAD - NCNC, TP), jnp.int32)], axis=1)
        s_pad = jnp.concatenate(
            [s_p, jnp.ones((CPAD - NCNC,), jnp.float32)])
        g0s.append(g_pad[:, :, 0:1])
        g1s.append(g_pad[:, :, 1:2])
        ss.append(s_pad.reshape(CPAD, 1))
    return (perm_out, jnp.stack(g0s), jnp.stack(g1s), jnp.stack(ss))


NWORK = 32
RPW = VOCAB // NWORK
GRAN = 128


def _sc_gather_kernel(table_hbm, idx_hbm, out_hbm, idx_v, rows_v, sem):
    wid = lax.axis_index("s") * 2 + lax.axis_index("c")
    base = wid * RPW
    pltpu.sync_copy(idx_hbm.at[pl.ds(base, RPW)], idx_v)
    pltpu.async_copy(table_hbm.at[idx_v], rows_v, sem).wait()
    pltpu.sync_copy(rows_v, out_hbm.at[pl.ds(base, RPW)])


def _sc_scatter_kernel(rows_hbm, idx_hbm, out_hbm, idx_v, rows_v, sem):
    wid = lax.axis_index("s") * 2 + lax.axis_index("c")
    base = wid * RPW
    pltpu.sync_copy(idx_hbm.at[pl.ds(base, RPW)], idx_v)
    pltpu.sync_copy(rows_hbm.at[pl.ds(base, RPW)], rows_v)
    pltpu.async_copy(rows_v, out_hbm.at[idx_v], sem).wait()


def _sc_call(body, n_out_rows):
    return pl.kernel(
        body,
        mesh=plsc.VectorSubcoreMesh(core_axis_name="c", subcore_axis_name="s"),
        out_type=jax.ShapeDtypeStruct((n_out_rows, GRAN), jnp.int32),
        scratch_types=[
            pltpu.VMEM((RPW,), jnp.int32),
            pltpu.VMEM((RPW, GRAN), jnp.int32),
            pltpu.SemaphoreType.DMA,
        ],
    )


def _logits_kernel(a_ref, out_ref):
    out_ref[...] = jax.lax.dot_general(
        a_ref[...], a_ref[...], (((1,), (1,)), ((), ())),
        preferred_element_type=jnp.float32)


def _exp_field(x_i32):
    y = (x_i32 + 1).astype(jnp.float32)
    return jax.lax.shift_right_arithmetic(
        jax.lax.bitcast_convert_type(y, jnp.int32), jnp.int32(23))


def _main_kernel(perm_ref, sta_smem, g0_ref, g1_ref, s_ref, logits_ref,
                 loct_in_ref, loc_out_ref, tl_ref,
                 loct, ap, spv, pend_cov, pend_upd, acc_tl):
    g = pl.program_id(0)
    rb = pl.program_id(1)

    @pl.when(jnp.logical_and(g == 0, rb == 0))
    def _init():
        loct[...] = loct_in_ref[...].astype(jnp.float32)
        acc_tl[...] = jnp.zeros_like(acc_tl)

    @pl.when(rb == 0)
    def _batch_start():
        ap[...] = jnp.abs(loct[...]).astype(jnp.int32)
        spv[...] = jnp.sign(loct[...]) * jnp.float32(1.0 / 16.0)
        pend_cov[...] = jnp.zeros_like(pend_cov)
        pend_upd[...] = jnp.zeros_like(pend_upd)

    sub_iota = jax.lax.broadcasted_iota(
        jnp.int32, (CPAD, 1), 0).astype(jnp.float32)
    lane_iota = jax.lax.broadcasted_iota(jnp.int32, (1, VOCAB), 1)
    s_col = s_ref[0]

    for rr in range(RB):
        r = rb * RB + rr
        idx = perm_ref[g, r]
        l0 = sta_smem[g, r, 0]
        l1 = sta_smem[g, r, 1]
        logit_row = logits_ref[pl.ds(idx, 1), :]
        ori = (jnp.abs(l0), jnp.abs(l1))
        sgn = (jnp.sign(l0).astype(jnp.float32),
               jnp.sign(l1).astype(jnp.float32))
        g_rows = (jnp.reshape(g0_ref[0, rr], (CPAD, 1)),
                  jnp.reshape(g1_ref[0, rr], (CPAD, 1)))
        lane_mask = (lane_iota == idx).astype(jnp.float32)
        pend_cov[...] += lane_mask
        for t in (0, 1):
            o = 1 - t
            ap_t = ap[pl.ds(t, 1), :]
            ap_o = ap[pl.ds(o, 1), :]
            spv_t = spv[pl.ds(t, 1), :]
            spv_o = spv[pl.ds(o, 1), :]
            e2 = _exp_field(ori[o] ^ ap_o)
            dsph = (134 - e2).astype(jnp.float32) * (spv_o * sgn[o])
            u = ori[t] ^ g_rows[t]
            gj = jnp.where(u == 0, jnp.float32(0.0), s_col)
            e = _exp_field(u ^ ap_t)
            dh = (134 - e).astype(jnp.float32) * (gj * spv_t)
            delt = (dh + dsph) - logit_row
            loss = jnp.sum(jnp.abs(delt), axis=1, keepdims=True)
            loss = jnp.where(sub_iota < NCNC, loss, jnp.float32(3.0e38))
            m = jnp.min(loss, axis=0, keepdims=True)
            jstar = jnp.min(jnp.where(loss == m, sub_iota, jnp.float32(1.0e9)),
                            axis=0, keepdims=True)
            candv = s_col * u.astype(jnp.float32)
            sel = jnp.sum(jnp.where(sub_iota == jstar, candv,
                                    jnp.float32(0.0)),
                          axis=0, keepdims=True)
            acc_tl[...] += m
            loc_out_ref[0, pl.ds(rr, 1), pl.ds(t, 1)] = sel.astype(jnp.int32)
            pend_upd[pl.ds(t, 1), :] += lane_mask * sel

    @pl.when(rb == BATCH // RB - 1)
    def _batch_end():
        loct[...] = (loct[...] * (jnp.float32(1.0) - pend_cov[...])
                     + pend_upd[...])

    @pl.when(jnp.logical_and(g == NB - 1, rb == BATCH // RB - 1))
    def _fin():
        tl_ref[...] = acc_tl[...] * jnp.float32(1.0 / (VOCAB * BATCH * TP * NB))


def kernel(eu_emb, locations):
    eu32 = eu_emb.astype(jnp.float32)
    loc32 = locations.astype(jnp.int32)
    loct_in = loc32.T
    with jax.ensure_compile_time_eval():
        perm_c, g0_c, g1_c, s_c = _rng_constants()
    perm_flat = perm_c.reshape(VOCAB)

    loc_pad = jnp.pad(loc32, ((0, 0), (0, GRAN - TP)))
    sta_pad = _sc_call(_sc_gather_kernel, VOCAB)(loc_pad, perm_flat)
    sta_c = sta_pad[:, :TP].reshape(NB, BATCH, TP)

    logits = pl.pallas_call(
        _logits_kernel,
        out_shape=jax.ShapeDtypeStruct((VOCAB, VOCAB), jnp.float32),
    )(eu32)

    grid_spec = pltpu.PrefetchScalarGridSpec(
        num_scalar_prefetch=0,
        grid=(NB, BATCH // RB),
        in_specs=[
            pl.BlockSpec((NB, BATCH), lambda g, r: (_i0(), _i0()),
                         memory_space=pltpu.SMEM),
            pl.BlockSpec((NB, BATCH, TP), lambda g, r: (_i0(), _i0(), _i0()),
                         memory_space=pltpu.SMEM),
            pl.BlockSpec((1, RB, CPAD, 1), lambda g, r: (g, r, _i0(), _i0())),
            pl.BlockSpec((1, RB, CPAD, 1), lambda g, r: (g, r, _i0(), _i0())),
            pl.BlockSpec((1, CPAD, 1), lambda g, r: (g, _i0(), _i0())),
            pl.BlockSpec((VOCAB, VOCAB), lambda g, r: (_i0(), _i0())),
            pl.BlockSpec((TP, VOCAB), lambda g, r: (_i0(), _i0())),
        ],
        out_specs=[
            pl.BlockSpec((1, RB, TP),
                         lambda g, r: (g * (BATCH // RB) + r, _i0(), _i0())),
            pl.BlockSpec((1, 1), lambda g, r: (_i0(), _i0())),
        ],
        scratch_shapes=[
            pltpu.VMEM((TP, VOCAB), jnp.float32),
            pltpu.VMEM((TP, VOCAB), jnp.int32),
            pltpu.VMEM((TP, VOCAB), jnp.float32),
            pltpu.VMEM((1, VOCAB), jnp.float32),
            pltpu.VMEM((TP, VOCAB), jnp.float32),
            pltpu.VMEM((1, 1), jnp.float32),
        ],
    )

    sel_seq, tl = pl.pallas_call(
        _main_kernel,
        grid_spec=grid_spec,
        out_shape=[
            jax.ShapeDtypeStruct((NB * BATCH // RB, RB, TP), jnp.int32),
            jax.ShapeDtypeStruct((1, 1), jnp.float32),
        ],
    )(perm_c, sta_c, g0_c, g1_c, s_c, logits, loct_in)

    sel_pad = jnp.pad(sel_seq.reshape(VOCAB, TP), ((0, 0), (0, GRAN - TP)))
    out_pad = _sc_call(_sc_scatter_kernel, VOCAB)(sel_pad, perm_flat)
    loc_out = out_pad[:, :TP]

    return loc_out.astype(locations.dtype), tl.reshape(()).astype(jnp.float32)
